# baseline (speedup 1.0000x reference)
import jax
import jax.numpy as jnp
from jax import lax
from jax.experimental import pallas as pl
from jax.experimental.pallas import tpu as pltpu
from jax.experimental.pallas import tpu_sc as plsc

N_NODES = 10000
N_EDGES = 320000
D = 128
D_E = 16
D_AUG = 32

NC = 2
NS = 16
NW = NC * NS
E_PER_W = N_EDGES // NW
CHUNK = 80
NCHUNK = E_PER_W // CHUNK

STRIPE = 640
LAST_STRIPE = N_NODES - 15 * STRIPE

BF = jnp.bfloat16


def _worker(cid, sid):
    return sid * NC + cid


def _idx_loader(idxb, hbm, ebase):
    def i_load(j):
        slot = lax.rem(j, 4)
        pltpu.sync_copy(hbm.at[pl.ds(ebase + j * CHUNK, CHUNK)],
                        idxb.at[slot])
    return i_load


def _stripe_zero(zsrc, table, sid):
    zbase = sid * STRIPE
    for t in range(STRIPE // CHUNK):
        @pl.when(zbase + t * CHUNK < N_NODES)
        def _():
            pltpu.sync_copy(zsrc, table.at[pl.ds(zbase + t * CHUNK, CHUNK)])


def _stripe_out(table, out, cid, sid):
    zbase = sid * STRIPE

    @pl.when(sid < NS - 1)
    def _():
        pltpu.sync_copy(table.at[pl.ds(zbase, STRIPE)],
                        out.at[cid, pl.ds(zbase, STRIPE)])

    @pl.when(sid == NS - 1)
    def _():
        pltpu.sync_copy(table.at[pl.ds(15 * STRIPE, LAST_STRIPE)],
                        out.at[cid, pl.ds(15 * STRIPE, LAST_STRIPE)])


DH = D // NC
E_PER_T = N_EDGES // NS
NCHUNK_G = E_PER_T // CHUNK


def _gather_body(nodes2_hbm, send_hbm, recv_hbm, out128,
                 sidxb, ridxb, rows0, rows1, gsem0, gsem1, s128):
    cid = lax.axis_index("c")
    sid = lax.axis_index("s")
    ebase = sid * E_PER_T
    rows = (rows0, rows1)
    gsems = (gsem0, gsem1)

    @pl.loop(0, CHUNK)
    def _z(i):
        for k in range(DH // 16):
            rows0[i, pl.ds(k * 16, 16)] = jnp.zeros((16,), jnp.float32)

    _stripe_zero(rows0, s128, sid)
    plsc.subcore_barrier()

    si_load = _idx_loader(sidxb, send_hbm, ebase)
    ri_load = _idx_loader(ridxb, recv_hbm, ebase)

    def g_start(j, b):
        slot = lax.rem(j, 4)
        return pltpu.async_copy(nodes2_hbm.at[cid].at[sidxb.at[slot]],
                                rows[b], gsems[b])

    def scatter(j, b):
        slot = lax.rem(j, 4)
        pltpu.sync_copy(rows[b], s128.at[ridxb.at[slot]], add=True)

    si_load(0)
    ri_load(0)
    si_load(1)
    ri_load(1)

    @pl.loop(0, NCHUNK_G, step=2)
    def _chunk(j):
        descs = [g_start(j, 0), g_start(j + 1, 1)]

        @pl.when(j + 2 < NCHUNK_G)
        def _():
            si_load(j + 2)
            ri_load(j + 2)

        @pl.when(j + 3 < NCHUNK_G)
        def _():
            si_load(j + 3)
            ri_load(j + 3)

        for b in range(2):
            descs[b].wait()
            scatter(j + b, b)

    plsc.subcore_barrier()
    _stripe_out(s128, out128, cid, sid)


_gather_call = pl.kernel(
    _gather_body,
    out_type=jax.ShapeDtypeStruct((NC, N_NODES, DH), jnp.float32),
    mesh=plsc.VectorSubcoreMesh(core_axis_name="c", subcore_axis_name="s"),
    scratch_types=[
        pltpu.VMEM((4, CHUNK), jnp.int32),
        pltpu.VMEM((4, CHUNK), jnp.int32),
        pltpu.VMEM((CHUNK, DH), jnp.float32),
        pltpu.VMEM((CHUNK, DH), jnp.float32),
        pltpu.SemaphoreType.DMA,
        pltpu.SemaphoreType.DMA,
        pltpu.VMEM_SHARED((N_NODES, DH), jnp.float32),
    ],
    compiler_params=pltpu.CompilerParams(use_tc_tiling_on_sc=False),
)


def _edge_body(eaug_hbm, recv_hbm, outaux,
               ridxb, eb0, eb1, esem0, esem1, saux):
    cid = lax.axis_index("c")
    sid = lax.axis_index("s")
    ebase = _worker(cid, sid) * E_PER_W
    ebs = (eb0, eb1)
    esems = (esem0, esem1)

    @pl.loop(0, CHUNK)
    def _z(i):
        eb0[i, pl.ds(0, 16)] = jnp.zeros((16,), jnp.float32)
        eb0[i, pl.ds(16, 16)] = jnp.zeros((16,), jnp.float32)

    _stripe_zero(eb0, saux, sid)
    plsc.subcore_barrier()

    ri_load = _idx_loader(ridxb, recv_hbm, ebase)

    def e_start(j, b):
        return pltpu.async_copy(eaug_hbm.at[pl.ds(ebase + j * CHUNK, CHUNK)],
                                ebs[b], esems[b])

    def scatter(j, b):
        slot = lax.rem(j, 4)
        pltpu.sync_copy(ebs[b], saux.at[ridxb.at[slot]], add=True)

    ri_load(0)
    ri_load(1)

    @pl.loop(0, NCHUNK - 1, step=2)
    def _chunk(j):
        descs = [e_start(j, 0), e_start(j + 1, 1)]

        @pl.when(j + 2 < NCHUNK)
        def _():
            ri_load(j + 2)

        @pl.when(j + 3 < NCHUNK)
        def _():
            ri_load(j + 3)

        for b in range(2):
            descs[b].wait()
            scatter(j + b, b)

    e_start(NCHUNK - 1, 0).wait()
    scatter(NCHUNK - 1, 0)

    plsc.subcore_barrier()
    _stripe_out(saux, outaux, cid, sid)


_edge_call = pl.kernel(
    _edge_body,
    out_type=jax.ShapeDtypeStruct((NC, N_NODES, D_AUG), jnp.float32),
    mesh=plsc.VectorSubcoreMesh(core_axis_name="c", subcore_axis_name="s"),
    scratch_types=[
        pltpu.VMEM((4, CHUNK), jnp.int32),
        pltpu.VMEM((CHUNK, D_AUG), jnp.float32),
        pltpu.VMEM((CHUNK, D_AUG), jnp.float32),
        pltpu.SemaphoreType.DMA,
        pltpu.SemaphoreType.DMA,
        pltpu.VMEM_SHARED((N_NODES, D_AUG), jnp.float32),
    ],
    compiler_params=pltpu.CompilerParams(use_tc_tiling_on_sc=False),
)


ROWS_TC = 1000


def _tc_body(eps_ref, nodes_ref, recv_ref, aux_ref,
             wbig_ref, w1_ref, b1_ref, w2_ref, b2_ref, out_ref):
    r = jnp.dot(aux_ref[...], wbig_ref[...],
                preferred_element_type=jnp.float32)
    h0 = (1.0 + eps_ref[0, 0]) * nodes_ref[...] + r + recv_ref[...]
    h1 = jnp.dot(h0, w1_ref[...], preferred_element_type=jnp.float32) + b1_ref[...]
    h1 = jnp.maximum(h1, 0.0)
    out_ref[...] = (jnp.dot(h1, w2_ref[...], preferred_element_type=jnp.float32)
                    + b2_ref[...])


_tc_call = pl.pallas_call(
    _tc_body,
    out_shape=jax.ShapeDtypeStruct((N_NODES, D), jnp.float32),
    grid=(N_NODES // ROWS_TC,),
    in_specs=[
        pl.BlockSpec((1, 1), lambda i: (0, 0)),
        pl.BlockSpec((ROWS_TC, D), lambda i: (i, 0)),
        pl.BlockSpec((ROWS_TC, D), lambda i: (i, 0)),
        pl.BlockSpec((ROWS_TC, D_AUG), lambda i: (i, 0)),
        pl.BlockSpec((D_AUG, D), lambda i: (0, 0)),
        pl.BlockSpec((D, D), lambda i: (0, 0)),
        pl.BlockSpec((1, D), lambda i: (0, 0)),
        pl.BlockSpec((D, D), lambda i: (0, 0)),
        pl.BlockSpec((1, D), lambda i: (0, 0)),
    ],
    out_specs=pl.BlockSpec((ROWS_TC, D), lambda i: (i, 0)),
)


@jax.jit
def _impl(nodes, edge_attr, senders, receivers, W_e, b_e, epsilon, W1, b1, W2, b2):
    send = senders.astype(jnp.int32)
    recv = receivers.astype(jnp.int32)
    nodes2 = jnp.stack([nodes[:, :DH], nodes[:, DH:]])
    eaug = jnp.concatenate(
        [edge_attr, jnp.ones((N_EDGES, D_AUG - D_E), jnp.float32)], axis=1)
    p128 = _gather_call(nodes2, send, recv)
    paux = _edge_call(eaug, recv)
    received = p128.transpose(1, 0, 2).reshape(N_NODES, D)
    aux = paux[0] + paux[1]
    wbig = jnp.concatenate(
        [W_e, b_e.reshape(1, D), jnp.zeros((D_AUG - D_E - 1, D), jnp.float32)],
        axis=0)
    return _tc_call(epsilon, nodes, received, aux,
                    wbig, W1, b1.reshape(1, D), W2, b2.reshape(1, D))


def kernel(nodes, edge_attr, senders, receivers, W_e, b_e, epsilon, W1, b1, W2, b2):
    return _impl(nodes, edge_attr, senders, receivers, W_e, b_e, epsilon,
                 W1, b1, W2, b2)


# speedup vs baseline: 1.1745x; 1.1745x over previous
import jax
import jax.numpy as jnp
from jax import lax
from jax.experimental import pallas as pl
from jax.experimental.pallas import tpu as pltpu
from jax.experimental.pallas import tpu_sc as plsc

N_NODES = 10000
N_EDGES = 320000
D = 128
D_E = 16
D_AUG = 32

NC = 2
NS = 16
NW = NC * NS
E_PER_W = N_EDGES // NW
CHUNK = 80
NCHUNK = E_PER_W // CHUNK

STRIPE = 640
LAST_STRIPE = N_NODES - 15 * STRIPE

BF = jnp.bfloat16


def _worker(cid, sid):
    return sid * NC + cid


def _idx_loader(idxb, hbm, ebase):
    def i_load(j):
        slot = lax.rem(j, 4)
        pltpu.sync_copy(hbm.at[pl.ds(ebase + j * CHUNK, CHUNK)],
                        idxb.at[slot])
    return i_load


def _stripe_zero(zsrc, table, sid):
    zbase = sid * STRIPE
    for t in range(STRIPE // CHUNK):
        @pl.when(zbase + t * CHUNK < N_NODES)
        def _():
            pltpu.sync_copy(zsrc, table.at[pl.ds(zbase + t * CHUNK, CHUNK)])


def _stripe_out(table, out, cid, sid):
    zbase = sid * STRIPE

    @pl.when(sid < NS - 1)
    def _():
        pltpu.sync_copy(table.at[pl.ds(zbase, STRIPE)],
                        out.at[cid, pl.ds(zbase, STRIPE)])

    @pl.when(sid == NS - 1)
    def _():
        pltpu.sync_copy(table.at[pl.ds(15 * STRIPE, LAST_STRIPE)],
                        out.at[cid, pl.ds(15 * STRIPE, LAST_STRIPE)])


DH = D // NC
NCHUNK_ALL = N_EDGES // CHUNK
TCHUNK = 256
LAST_TCHUNK = NCHUNK_ALL - 15 * TCHUNK
GRP = 8


def _gather_body(nodes2_hbm, send2_hbm, recv2_hbm, out128,
                 sidxb, ridxb, rows0, rows1, gsem0, gsem1, ssem0, ssem1,
                 s128):
    cid = lax.axis_index("c")
    sid = lax.axis_index("s")
    tbase = sid * TCHUNK
    nchunk = jnp.where(sid < NS - 1, TCHUNK, LAST_TCHUNK)
    rows = (rows0, rows1)
    gsems = (gsem0, gsem1)
    ssems = (ssem0, ssem1)

    @pl.loop(0, CHUNK)
    def _z(i):
        for k in range(DH // 16):
            rows0[i, pl.ds(k * 16, 16)] = jnp.zeros((16,), jnp.float32)

    _stripe_zero(rows0, s128, sid)
    plsc.subcore_barrier()

    def i_group_load(j):
        pltpu.sync_copy(send2_hbm.at[pl.ds(tbase + j, GRP)], sidxb)
        pltpu.sync_copy(recv2_hbm.at[pl.ds(tbase + j, GRP)], ridxb)

    def g_desc(j, b):
        slot = lax.rem(j, GRP)
        return pltpu.make_async_copy(
            nodes2_hbm.at[cid].at[sidxb.at[slot]], rows[b], gsems[b])

    def s_desc(j, b):
        slot = lax.rem(j, GRP)
        return pltpu.make_async_copy(rows[b], s128.at[ridxb.at[slot]],
                                     ssems[b])

    def g_start(j, b):
        slot = lax.rem(j, GRP)
        pltpu.async_copy(nodes2_hbm.at[cid].at[sidxb.at[slot]], rows[b],
                         gsems[b])

    def s_start(j, b):
        slot = lax.rem(j, GRP)
        pltpu.async_copy(rows[b], s128.at[ridxb.at[slot]], ssems[b],
                         add=True)

    i_group_load(0)
    g_start(0, 0)
    g_start(1, 1)

    @pl.loop(0, nchunk, step=GRP)
    def _group(g0):
        for q in range(GRP // 2):
            c0 = g0 + 2 * q
            c1 = c0 + 1
            g_desc(c0, 0).wait()
            s_start(c0, 0)
            g_desc(c1, 1).wait()
            s_start(c1, 1)
            if q < GRP // 2 - 1:
                s_desc(c0, 0).wait()
                g_start(c0 + 2, 0)
                s_desc(c1, 1).wait()
                g_start(c1 + 2, 1)
            else:
                s_desc(c0, 0).wait()
                s_desc(c1, 1).wait()

                @pl.when(g0 + GRP < nchunk)
                def _():
                    i_group_load(g0 + GRP)
                    g_start(c0 + 2, 0)
                    g_start(c1 + 2, 1)

    plsc.subcore_barrier()
    _stripe_out(s128, out128, cid, sid)


_gather_call = pl.kernel(
    _gather_body,
    out_type=jax.ShapeDtypeStruct((NC, N_NODES, DH), jnp.float32),
    mesh=plsc.VectorSubcoreMesh(core_axis_name="c", subcore_axis_name="s"),
    scratch_types=[
        pltpu.VMEM((GRP, CHUNK), jnp.int32),
        pltpu.VMEM((GRP, CHUNK), jnp.int32),
        pltpu.VMEM((CHUNK, DH), jnp.float32),
        pltpu.VMEM((CHUNK, DH), jnp.float32),
        pltpu.SemaphoreType.DMA,
        pltpu.SemaphoreType.DMA,
        pltpu.SemaphoreType.DMA,
        pltpu.SemaphoreType.DMA,
        pltpu.VMEM_SHARED((N_NODES, DH), jnp.float32),
    ],
    compiler_params=pltpu.CompilerParams(use_tc_tiling_on_sc=False),
)


def _edge_body(eaug_hbm, recv_hbm, outaux,
               ridxb, eb0, eb1, esem0, esem1, saux):
    cid = lax.axis_index("c")
    sid = lax.axis_index("s")
    ebase = _worker(cid, sid) * E_PER_W
    ebs = (eb0, eb1)
    esems = (esem0, esem1)

    @pl.loop(0, CHUNK)
    def _z(i):
        eb0[i, pl.ds(0, 16)] = jnp.zeros((16,), jnp.float32)
        eb0[i, pl.ds(16, 16)] = jnp.zeros((16,), jnp.float32)

    _stripe_zero(eb0, saux, sid)
    plsc.subcore_barrier()

    ri_load = _idx_loader(ridxb, recv_hbm, ebase)

    def e_start(j, b):
        return pltpu.async_copy(eaug_hbm.at[pl.ds(ebase + j * CHUNK, CHUNK)],
                                ebs[b], esems[b])

    def scatter(j, b):
        slot = lax.rem(j, 4)
        pltpu.sync_copy(ebs[b], saux.at[ridxb.at[slot]], add=True)

    ri_load(0)
    ri_load(1)

    @pl.loop(0, NCHUNK - 1, step=2)
    def _chunk(j):
        descs = [e_start(j, 0), e_start(j + 1, 1)]

        @pl.when(j + 2 < NCHUNK)
        def _():
            ri_load(j + 2)

        @pl.when(j + 3 < NCHUNK)
        def _():
            ri_load(j + 3)

        for b in range(2):
            descs[b].wait()
            scatter(j + b, b)

    e_start(NCHUNK - 1, 0).wait()
    scatter(NCHUNK - 1, 0)

    plsc.subcore_barrier()
    _stripe_out(saux, outaux, cid, sid)


_edge_call = pl.kernel(
    _edge_body,
    out_type=jax.ShapeDtypeStruct((NC, N_NODES, D_AUG), jnp.float32),
    mesh=plsc.VectorSubcoreMesh(core_axis_name="c", subcore_axis_name="s"),
    scratch_types=[
        pltpu.VMEM((4, CHUNK), jnp.int32),
        pltpu.VMEM((CHUNK, D_AUG), jnp.float32),
        pltpu.VMEM((CHUNK, D_AUG), jnp.float32),
        pltpu.SemaphoreType.DMA,
        pltpu.SemaphoreType.DMA,
        pltpu.VMEM_SHARED((N_NODES, D_AUG), jnp.float32),
    ],
    compiler_params=pltpu.CompilerParams(use_tc_tiling_on_sc=False),
)


ROWS_TC = 1000


def _tc_body(eps_ref, nodes_ref, recv_ref, aux_ref,
             wbig_ref, w1_ref, b1_ref, w2_ref, b2_ref, out_ref):
    r = jnp.dot(aux_ref[...], wbig_ref[...],
                preferred_element_type=jnp.float32)
    h0 = (1.0 + eps_ref[0, 0]) * nodes_ref[...] + r + recv_ref[...]
    h1 = jnp.dot(h0, w1_ref[...], preferred_element_type=jnp.float32) + b1_ref[...]
    h1 = jnp.maximum(h1, 0.0)
    out_ref[...] = (jnp.dot(h1, w2_ref[...], preferred_element_type=jnp.float32)
                    + b2_ref[...])


_tc_call = pl.pallas_call(
    _tc_body,
    out_shape=jax.ShapeDtypeStruct((N_NODES, D), jnp.float32),
    grid=(N_NODES // ROWS_TC,),
    in_specs=[
        pl.BlockSpec((1, 1), lambda i: (0, 0)),
        pl.BlockSpec((ROWS_TC, D), lambda i: (i, 0)),
        pl.BlockSpec((ROWS_TC, D), lambda i: (i, 0)),
        pl.BlockSpec((ROWS_TC, D_AUG), lambda i: (i, 0)),
        pl.BlockSpec((D_AUG, D), lambda i: (0, 0)),
        pl.BlockSpec((D, D), lambda i: (0, 0)),
        pl.BlockSpec((1, D), lambda i: (0, 0)),
        pl.BlockSpec((D, D), lambda i: (0, 0)),
        pl.BlockSpec((1, D), lambda i: (0, 0)),
    ],
    out_specs=pl.BlockSpec((ROWS_TC, D), lambda i: (i, 0)),
)


@jax.jit
def _impl(nodes, edge_attr, senders, receivers, W_e, b_e, epsilon, W1, b1, W2, b2):
    send = senders.astype(jnp.int32)
    recv = receivers.astype(jnp.int32)
    nodes2 = jnp.stack([nodes[:, :DH], nodes[:, DH:]])
    eaug = jnp.concatenate(
        [edge_attr, jnp.ones((N_EDGES, D_AUG - D_E), jnp.float32)], axis=1)
    send2 = send.reshape(NCHUNK_ALL, CHUNK)
    recv2 = recv.reshape(NCHUNK_ALL, CHUNK)
    p128 = _gather_call(nodes2, send2, recv2)
    paux = _edge_call(eaug, recv)
    received = p128.transpose(1, 0, 2).reshape(N_NODES, D)
    aux = paux[0] + paux[1]
    wbig = jnp.concatenate(
        [W_e, b_e.reshape(1, D), jnp.zeros((D_AUG - D_E - 1, D), jnp.float32)],
        axis=0)
    return _tc_call(epsilon, nodes, received, aux,
                    wbig, W1, b1.reshape(1, D), W2, b2.reshape(1, D))


def kernel(nodes, edge_attr, senders, receivers, W_e, b_e, epsilon, W1, b1, W2, b2):
    return _impl(nodes, edge_attr, senders, receivers, W_e, b_e, epsilon,
                 W1, b1, W2, b2)


# speedup vs baseline: 1.3846x; 1.1789x over previous
import jax
import jax.numpy as jnp
from jax import lax
from jax.experimental import pallas as pl
from jax.experimental.pallas import tpu as pltpu
from jax.experimental.pallas import tpu_sc as plsc

N_NODES = 10000
N_EDGES = 320000
D = 128
D_E = 16
D_AUG = 32

NC = 2
NS = 16
NW = NC * NS
E_PER_W = N_EDGES // NW
CHUNK = 80
NCHUNK = E_PER_W // CHUNK

STRIPE = 640
LAST_STRIPE = N_NODES - 15 * STRIPE

BF = jnp.bfloat16


def _worker(cid, sid):
    return sid * NC + cid


def _idx_loader(idxb, hbm, ebase):
    def i_load(j):
        slot = lax.rem(j, 4)
        pltpu.sync_copy(hbm.at[pl.ds(ebase + j * CHUNK, CHUNK)],
                        idxb.at[slot])
    return i_load


def _stripe_zero(zsrc, table, sid):
    zbase = sid * STRIPE
    for t in range(STRIPE // CHUNK):
        @pl.when(zbase + t * CHUNK < N_NODES)
        def _():
            pltpu.sync_copy(zsrc, table.at[pl.ds(zbase + t * CHUNK, CHUNK)])


def _stripe_out(table, out, cid, sid):
    zbase = sid * STRIPE

    @pl.when(sid < NS - 1)
    def _():
        pltpu.sync_copy(table.at[pl.ds(zbase, STRIPE)],
                        out.at[cid, pl.ds(zbase, STRIPE)])

    @pl.when(sid == NS - 1)
    def _():
        pltpu.sync_copy(table.at[pl.ds(15 * STRIPE, LAST_STRIPE)],
                        out.at[cid, pl.ds(15 * STRIPE, LAST_STRIPE)])


DH = D // NC
NCHUNK_ALL = N_EDGES // CHUNK
TCHUNK = 256
LAST_TCHUNK = NCHUNK_ALL - 15 * TCHUNK
GRP = 8


def _gather_body(nodes2_hbm, send2_hbm, recv2_hbm, out128,
                 sidxb, ridxb, rows0, rows1, gsem0, gsem1, ssem0, ssem1,
                 s128):
    cid = lax.axis_index("c")
    sid = lax.axis_index("s")
    tbase = sid * TCHUNK
    nchunk = jnp.where(sid < NS - 1, TCHUNK, LAST_TCHUNK)
    rows = (rows0, rows1)
    gsems = (gsem0, gsem1)
    ssems = (ssem0, ssem1)

    @pl.loop(0, CHUNK)
    def _z(i):
        for k in range(DH // 16):
            rows0[i, pl.ds(k * 16, 16)] = jnp.zeros((16,), jnp.float32)

    _stripe_zero(rows0, s128, sid)
    plsc.subcore_barrier()

    def i_group_load(j):
        pltpu.sync_copy(send2_hbm.at[pl.ds(tbase + j, GRP)], sidxb)
        pltpu.sync_copy(recv2_hbm.at[pl.ds(tbase + j, GRP)], ridxb)

    def g_desc(j, b):
        slot = lax.rem(j, GRP)
        return pltpu.make_async_copy(
            nodes2_hbm.at[cid].at[sidxb.at[slot]], rows[b], gsems[b])

    def s_desc(j, b):
        slot = lax.rem(j, GRP)
        return pltpu.make_async_copy(rows[b], s128.at[ridxb.at[slot]],
                                     ssems[b])

    def g_start(j, b):
        slot = lax.rem(j, GRP)
        pltpu.async_copy(nodes2_hbm.at[cid].at[sidxb.at[slot]], rows[b],
                         gsems[b])

    def s_start(j, b):
        slot = lax.rem(j, GRP)
        pltpu.async_copy(rows[b], s128.at[ridxb.at[slot]], ssems[b],
                         add=True)

    i_group_load(0)
    g_start(0, 0)
    g_start(1, 1)

    @pl.loop(0, nchunk, step=GRP)
    def _group(g0):
        for q in range(GRP // 2):
            c0 = g0 + 2 * q
            c1 = c0 + 1
            g_desc(c0, 0).wait()
            s_start(c0, 0)
            g_desc(c1, 1).wait()
            s_start(c1, 1)
            if q < GRP // 2 - 1:
                s_desc(c0, 0).wait()
                g_start(c0 + 2, 0)
                s_desc(c1, 1).wait()
                g_start(c1 + 2, 1)
            else:
                s_desc(c0, 0).wait()
                s_desc(c1, 1).wait()

                @pl.when(g0 + GRP < nchunk)
                def _():
                    i_group_load(g0 + GRP)
                    g_start(c0 + 2, 0)
                    g_start(c1 + 2, 1)

    plsc.subcore_barrier()
    _stripe_out(s128, out128, cid, sid)


_gather_call = pl.kernel(
    _gather_body,
    out_type=jax.ShapeDtypeStruct((NC, N_NODES, DH), jnp.float32),
    mesh=plsc.VectorSubcoreMesh(core_axis_name="c", subcore_axis_name="s"),
    scratch_types=[
        pltpu.VMEM((GRP, CHUNK), jnp.int32),
        pltpu.VMEM((GRP, CHUNK), jnp.int32),
        pltpu.VMEM((CHUNK, DH), jnp.float32),
        pltpu.VMEM((CHUNK, DH), jnp.float32),
        pltpu.SemaphoreType.DMA,
        pltpu.SemaphoreType.DMA,
        pltpu.SemaphoreType.DMA,
        pltpu.SemaphoreType.DMA,
        pltpu.VMEM_SHARED((N_NODES, DH), jnp.float32),
    ],
    compiler_params=pltpu.CompilerParams(use_tc_tiling_on_sc=False),
)


ECHUNK_C = NCHUNK_ALL // NC
ETCHUNK = 128
LAST_ETCHUNK = ECHUNK_C - 15 * ETCHUNK


def _edge_body(eattr_hbm, recv2_hbm, out16, outcnt,
               ridxb, eb0, eb1, ones_b, esem0, esem1, fsem0, fsem1,
               csem0, csem1, s16, scnt):
    cid = lax.axis_index("c")
    sid = lax.axis_index("s")
    tbase = cid * ECHUNK_C + sid * ETCHUNK
    nchunk = jnp.where(sid < NS - 1, ETCHUNK, LAST_ETCHUNK)
    ebs = (eb0, eb1)
    esems = (esem0, esem1)
    fsems = (fsem0, fsem1)
    csems = (csem0, csem1)

    @pl.loop(0, CHUNK)
    def _z(i):
        eb0[i, :] = jnp.zeros((16,), jnp.float32)
        ones_b[i, :] = jnp.full((16,), 1.0, jnp.float32)

    _stripe_zero(eb0, s16, sid)
    _stripe_zero(eb0, scnt, sid)
    plsc.subcore_barrier()

    def i_group_load(j):
        pltpu.sync_copy(recv2_hbm.at[pl.ds(tbase + j, GRP)], ridxb)

    def e_desc(j, b):
        return pltpu.make_async_copy(
            eattr_hbm.at[pl.ds((tbase + j) * CHUNK, CHUNK)], ebs[b],
            esems[b])

    def e_start(j, b):
        pltpu.async_copy(eattr_hbm.at[pl.ds((tbase + j) * CHUNK, CHUNK)],
                         ebs[b], esems[b])

    def s_start(j, b):
        slot = lax.rem(j, GRP)
        pltpu.async_copy(ebs[b], s16.at[ridxb.at[slot]], fsems[b], add=True)
        pltpu.async_copy(ones_b, scnt.at[ridxb.at[slot]], csems[b], add=True)

    def s_wait(j, b):
        slot = lax.rem(j, GRP)
        pltpu.make_async_copy(ebs[b], s16.at[ridxb.at[slot]],
                              fsems[b]).wait()
        pltpu.make_async_copy(ones_b, scnt.at[ridxb.at[slot]],
                              csems[b]).wait()

    i_group_load(0)
    e_start(0, 0)
    e_start(1, 1)

    @pl.loop(0, nchunk, step=GRP)
    def _group(g0):
        for q in range(GRP // 2):
            c0 = g0 + 2 * q
            c1 = c0 + 1
            e_desc(c0, 0).wait()
            s_start(c0, 0)
            e_desc(c1, 1).wait()
            s_start(c1, 1)
            if q < GRP // 2 - 1:
                s_wait(c0, 0)
                e_start(c0 + 2, 0)
                s_wait(c1, 1)
                e_start(c1 + 2, 1)
            else:
                s_wait(c0, 0)
                s_wait(c1, 1)

                @pl.when(g0 + GRP < nchunk)
                def _():
                    i_group_load(g0 + GRP)
                    e_start(c0 + 2, 0)
                    e_start(c1 + 2, 1)

    plsc.subcore_barrier()
    _stripe_out(s16, out16, cid, sid)
    _stripe_out(scnt, outcnt, cid, sid)


_edge_call = pl.kernel(
    _edge_body,
    out_type=(
        jax.ShapeDtypeStruct((NC, N_NODES, D_E), jnp.float32),
        jax.ShapeDtypeStruct((NC, N_NODES, D_E), jnp.float32),
    ),
    mesh=plsc.VectorSubcoreMesh(core_axis_name="c", subcore_axis_name="s"),
    scratch_types=[
        pltpu.VMEM((GRP, CHUNK), jnp.int32),
        pltpu.VMEM((CHUNK, D_E), jnp.float32),
        pltpu.VMEM((CHUNK, D_E), jnp.float32),
        pltpu.VMEM((CHUNK, D_E), jnp.float32),
        pltpu.SemaphoreType.DMA,
        pltpu.SemaphoreType.DMA,
        pltpu.SemaphoreType.DMA,
        pltpu.SemaphoreType.DMA,
        pltpu.SemaphoreType.DMA,
        pltpu.SemaphoreType.DMA,
        pltpu.VMEM_SHARED((N_NODES, D_E), jnp.float32),
        pltpu.VMEM_SHARED((N_NODES, D_E), jnp.float32),
    ],
    compiler_params=pltpu.CompilerParams(use_tc_tiling_on_sc=False),
)


ROWS_TC = 1000


def _tc_body(eps_ref, nodes_ref, recv_ref, aux_ref,
             wbig_ref, w1_ref, b1_ref, w2_ref, b2_ref, out_ref):
    r = jnp.dot(aux_ref[...], wbig_ref[...],
                preferred_element_type=jnp.float32)
    h0 = (1.0 + eps_ref[0, 0]) * nodes_ref[...] + r + recv_ref[...]
    h1 = jnp.dot(h0, w1_ref[...], preferred_element_type=jnp.float32) + b1_ref[...]
    h1 = jnp.maximum(h1, 0.0)
    out_ref[...] = (jnp.dot(h1, w2_ref[...], preferred_element_type=jnp.float32)
                    + b2_ref[...])


_tc_call = pl.pallas_call(
    _tc_body,
    out_shape=jax.ShapeDtypeStruct((N_NODES, D), jnp.float32),
    grid=(N_NODES // ROWS_TC,),
    in_specs=[
        pl.BlockSpec((1, 1), lambda i: (0, 0)),
        pl.BlockSpec((ROWS_TC, D), lambda i: (i, 0)),
        pl.BlockSpec((ROWS_TC, D), lambda i: (i, 0)),
        pl.BlockSpec((ROWS_TC, D_AUG), lambda i: (i, 0)),
        pl.BlockSpec((D_AUG, D), lambda i: (0, 0)),
        pl.BlockSpec((D, D), lambda i: (0, 0)),
        pl.BlockSpec((1, D), lambda i: (0, 0)),
        pl.BlockSpec((D, D), lambda i: (0, 0)),
        pl.BlockSpec((1, D), lambda i: (0, 0)),
    ],
    out_specs=pl.BlockSpec((ROWS_TC, D), lambda i: (i, 0)),
)


@jax.jit
def _impl(nodes, edge_attr, senders, receivers, W_e, b_e, epsilon, W1, b1, W2, b2):
    send = senders.astype(jnp.int32)
    recv = receivers.astype(jnp.int32)
    nodes2 = jnp.stack([nodes[:, :DH], nodes[:, DH:]])
    send2 = send.reshape(NCHUNK_ALL, CHUNK)
    recv2 = recv.reshape(NCHUNK_ALL, CHUNK)
    p128 = _gather_call(nodes2, send2, recv2)
    p16, pcnt = _edge_call(edge_attr, recv2)
    received = p128.transpose(1, 0, 2).reshape(N_NODES, D)
    aux = jnp.concatenate([p16[0] + p16[1], pcnt[0] + pcnt[1]], axis=1)
    wbig = jnp.concatenate(
        [W_e, b_e.reshape(1, D), jnp.zeros((D_AUG - D_E - 1, D), jnp.float32)],
        axis=0)
    return _tc_call(epsilon, nodes, received, aux,
                    wbig, W1, b1.reshape(1, D), W2, b2.reshape(1, D))


def kernel(nodes, edge_attr, senders, receivers, W_e, b_e, epsilon, W1, b1, W2, b2):
    return _impl(nodes, edge_attr, senders, receivers, W_e, b_e, epsilon,
                 W1, b1, W2, b2)


# speedup vs baseline: 1.3858x; 1.0009x over previous
import jax
import jax.numpy as jnp
from jax import lax
from jax.experimental import pallas as pl
from jax.experimental.pallas import tpu as pltpu
from jax.experimental.pallas import tpu_sc as plsc

N_NODES = 10000
N_EDGES = 320000
D = 128
D_E = 16
D_AUG = 32

NC = 2
NS = 16
CHUNK = 80

STRIPE = 640
LAST_STRIPE = N_NODES - 15 * STRIPE


def _stripe_zero(zsrc, table, sid):
    zbase = sid * STRIPE
    for t in range(STRIPE // CHUNK):
        @pl.when(zbase + t * CHUNK < N_NODES)
        def _():
            pltpu.sync_copy(zsrc, table.at[pl.ds(zbase + t * CHUNK, CHUNK)])


def _stripe_out(table, out, cid, sid):
    zbase = sid * STRIPE

    @pl.when(sid < NS - 1)
    def _():
        pltpu.sync_copy(table.at[pl.ds(zbase, STRIPE)],
                        out.at[cid, pl.ds(zbase, STRIPE)])

    @pl.when(sid == NS - 1)
    def _():
        pltpu.sync_copy(table.at[pl.ds(15 * STRIPE, LAST_STRIPE)],
                        out.at[cid, pl.ds(15 * STRIPE, LAST_STRIPE)])


DH = D // NC
NCHUNK_ALL = N_EDGES // CHUNK
TCHUNK = 256
LAST_TCHUNK = NCHUNK_ALL - 15 * TCHUNK
GRP = 8


def _gather_body(nodes2_hbm, send2_hbm, recv2_hbm, out128,
                 sidxb, ridxb, rows0, rows1, gsem0, gsem1, ssem0, ssem1,
                 s128):
    cid = lax.axis_index("c")
    sid = lax.axis_index("s")
    tbase = sid * TCHUNK
    nchunk = jnp.where(sid < NS - 1, TCHUNK, LAST_TCHUNK)
    rows = (rows0, rows1)
    gsems = (gsem0, gsem1)
    ssems = (ssem0, ssem1)

    @pl.loop(0, CHUNK)
    def _z(i):
        for k in range(DH // 16):
            rows0[i, pl.ds(k * 16, 16)] = jnp.zeros((16,), jnp.float32)

    _stripe_zero(rows0, s128, sid)
    plsc.subcore_barrier()

    def i_group_load(j):
        pltpu.sync_copy(send2_hbm.at[pl.ds(tbase + j, GRP)], sidxb)
        pltpu.sync_copy(recv2_hbm.at[pl.ds(tbase + j, GRP)], ridxb)

    def g_desc(j, b):
        slot = lax.rem(j, GRP)
        return pltpu.make_async_copy(
            nodes2_hbm.at[cid].at[sidxb.at[slot]], rows[b], gsems[b])

    def s_desc(j, b):
        slot = lax.rem(j, GRP)
        return pltpu.make_async_copy(rows[b], s128.at[ridxb.at[slot]],
                                     ssems[b])

    def g_start(j, b):
        slot = lax.rem(j, GRP)
        pltpu.async_copy(nodes2_hbm.at[cid].at[sidxb.at[slot]], rows[b],
                         gsems[b])

    def s_start(j, b):
        slot = lax.rem(j, GRP)
        pltpu.async_copy(rows[b], s128.at[ridxb.at[slot]], ssems[b],
                         add=True)

    i_group_load(0)
    g_start(0, 0)
    g_start(1, 1)

    @pl.loop(0, nchunk, step=GRP)
    def _group(g0):
        for q in range(GRP // 2):
            c0 = g0 + 2 * q
            c1 = c0 + 1
            g_desc(c0, 0).wait()
            s_start(c0, 0)
            g_desc(c1, 1).wait()
            s_start(c1, 1)
            if q < GRP // 2 - 1:
                s_desc(c0, 0).wait()
                g_start(c0 + 2, 0)
                s_desc(c1, 1).wait()
                g_start(c1 + 2, 1)
            else:
                s_desc(c0, 0).wait()
                s_desc(c1, 1).wait()

                @pl.when(g0 + GRP < nchunk)
                def _():
                    i_group_load(g0 + GRP)
                    g_start(c0 + 2, 0)
                    g_start(c1 + 2, 1)

    plsc.subcore_barrier()
    _stripe_out(s128, out128, cid, sid)


_gather_call = pl.kernel(
    _gather_body,
    out_type=jax.ShapeDtypeStruct((NC, N_NODES, DH), jnp.float32),
    mesh=plsc.VectorSubcoreMesh(core_axis_name="c", subcore_axis_name="s"),
    scratch_types=[
        pltpu.VMEM((GRP, CHUNK), jnp.int32),
        pltpu.VMEM((GRP, CHUNK), jnp.int32),
        pltpu.VMEM((CHUNK, DH), jnp.float32),
        pltpu.VMEM((CHUNK, DH), jnp.float32),
        pltpu.SemaphoreType.DMA,
        pltpu.SemaphoreType.DMA,
        pltpu.SemaphoreType.DMA,
        pltpu.SemaphoreType.DMA,
        pltpu.VMEM_SHARED((N_NODES, DH), jnp.float32),
    ],
    compiler_params=pltpu.CompilerParams(use_tc_tiling_on_sc=False),
)


ECHUNK_C = NCHUNK_ALL // NC
ETCHUNK = 128
LAST_ETCHUNK = ECHUNK_C - 15 * ETCHUNK


def _edge_body(eattr_hbm, recv2_hbm, out16, outcnt,
               ridxb, eb0, eb1, ones_b, esem0, esem1, fsem0, fsem1,
               csem0, csem1, s16, scnt):
    cid = lax.axis_index("c")
    sid = lax.axis_index("s")
    tbase = cid * ECHUNK_C + sid * ETCHUNK
    nchunk = jnp.where(sid < NS - 1, ETCHUNK, LAST_ETCHUNK)
    ebs = (eb0, eb1)
    esems = (esem0, esem1)
    fsems = (fsem0, fsem1)
    csems = (csem0, csem1)

    @pl.loop(0, CHUNK)
    def _z(i):
        eb0[i, :] = jnp.zeros((16,), jnp.float32)
        ones_b[i, :] = jnp.full((16,), 1.0, jnp.float32)

    _stripe_zero(eb0, s16, sid)
    _stripe_zero(eb0, scnt, sid)
    plsc.subcore_barrier()

    def i_group_load(j):
        pltpu.sync_copy(recv2_hbm.at[pl.ds(tbase + j, GRP)], ridxb)

    def e_desc(j, b):
        return pltpu.make_async_copy(
            eattr_hbm.at[pl.ds((tbase + j) * CHUNK, CHUNK)], ebs[b],
            esems[b])

    def e_start(j, b):
        pltpu.async_copy(eattr_hbm.at[pl.ds((tbase + j) * CHUNK, CHUNK)],
                         ebs[b], esems[b])

    def s_start(j, b):
        slot = lax.rem(j, GRP)
        pltpu.async_copy(ebs[b], s16.at[ridxb.at[slot]], fsems[b], add=True)
        pltpu.async_copy(ones_b, scnt.at[ridxb.at[slot]], csems[b], add=True)

    def s_wait(j, b):
        slot = lax.rem(j, GRP)
        pltpu.make_async_copy(ebs[b], s16.at[ridxb.at[slot]],
                              fsems[b]).wait()
        pltpu.make_async_copy(ones_b, scnt.at[ridxb.at[slot]],
                              csems[b]).wait()

    i_group_load(0)
    e_start(0, 0)
    e_start(1, 1)

    @pl.loop(0, nchunk, step=GRP)
    def _group(g0):
        for q in range(GRP // 2):
            c0 = g0 + 2 * q
            c1 = c0 + 1
            e_desc(c0, 0).wait()
            s_start(c0, 0)
            e_desc(c1, 1).wait()
            s_start(c1, 1)
            if q < GRP // 2 - 1:
                s_wait(c0, 0)
                e_start(c0 + 2, 0)
                s_wait(c1, 1)
                e_start(c1 + 2, 1)
            else:
                s_wait(c0, 0)
                s_wait(c1, 1)

                @pl.when(g0 + GRP < nchunk)
                def _():
                    i_group_load(g0 + GRP)
                    e_start(c0 + 2, 0)
                    e_start(c1 + 2, 1)

    plsc.subcore_barrier()
    _stripe_out(s16, out16, cid, sid)
    _stripe_out(scnt, outcnt, cid, sid)


_edge_call = pl.kernel(
    _edge_body,
    out_type=(
        jax.ShapeDtypeStruct((NC, N_NODES, D_E), jnp.float32),
        jax.ShapeDtypeStruct((NC, N_NODES, D_E), jnp.float32),
    ),
    mesh=plsc.VectorSubcoreMesh(core_axis_name="c", subcore_axis_name="s"),
    scratch_types=[
        pltpu.VMEM((GRP, CHUNK), jnp.int32),
        pltpu.VMEM((CHUNK, D_E), jnp.float32),
        pltpu.VMEM((CHUNK, D_E), jnp.float32),
        pltpu.VMEM((CHUNK, D_E), jnp.float32),
        pltpu.SemaphoreType.DMA,
        pltpu.SemaphoreType.DMA,
        pltpu.SemaphoreType.DMA,
        pltpu.SemaphoreType.DMA,
        pltpu.SemaphoreType.DMA,
        pltpu.SemaphoreType.DMA,
        pltpu.VMEM_SHARED((N_NODES, D_E), jnp.float32),
        pltpu.VMEM_SHARED((N_NODES, D_E), jnp.float32),
    ],
    compiler_params=pltpu.CompilerParams(use_tc_tiling_on_sc=False),
)


ROWS_TC = 1000


def _tc_body(eps_ref, nodes_ref, recv_ref, aux_ref,
             wbig_ref, w1_ref, b1_ref, w2_ref, b2_ref, out_ref):
    r = jnp.dot(aux_ref[...], wbig_ref[...],
                preferred_element_type=jnp.float32)
    h0 = (1.0 + eps_ref[0, 0]) * nodes_ref[...] + r + recv_ref[...]
    h1 = jnp.dot(h0, w1_ref[...], preferred_element_type=jnp.float32) + b1_ref[...]
    h1 = jnp.maximum(h1, 0.0)
    out_ref[...] = (jnp.dot(h1, w2_ref[...], preferred_element_type=jnp.float32)
                    + b2_ref[...])


_tc_call = pl.pallas_call(
    _tc_body,
    out_shape=jax.ShapeDtypeStruct((N_NODES, D), jnp.float32),
    grid=(N_NODES // ROWS_TC,),
    in_specs=[
        pl.BlockSpec((1, 1), lambda i: (0, 0)),
        pl.BlockSpec((ROWS_TC, D), lambda i: (i, 0)),
        pl.BlockSpec((ROWS_TC, D), lambda i: (i, 0)),
        pl.BlockSpec((ROWS_TC, D_AUG), lambda i: (i, 0)),
        pl.BlockSpec((D_AUG, D), lambda i: (0, 0)),
        pl.BlockSpec((D, D), lambda i: (0, 0)),
        pl.BlockSpec((1, D), lambda i: (0, 0)),
        pl.BlockSpec((D, D), lambda i: (0, 0)),
        pl.BlockSpec((1, D), lambda i: (0, 0)),
    ],
    out_specs=pl.BlockSpec((ROWS_TC, D), lambda i: (i, 0)),
)


@jax.jit
def _impl(nodes, edge_attr, senders, receivers, W_e, b_e, epsilon, W1, b1, W2, b2):
    send = senders.astype(jnp.int32)
    recv = receivers.astype(jnp.int32)
    nodes2 = jnp.stack([nodes[:, :DH], nodes[:, DH:]])
    send2 = send.reshape(NCHUNK_ALL, CHUNK)
    recv2 = recv.reshape(NCHUNK_ALL, CHUNK)
    p128 = _gather_call(nodes2, send2, recv2)
    p16, pcnt = _edge_call(edge_attr, recv2)
    received = p128.transpose(1, 0, 2).reshape(N_NODES, D)
    aux = jnp.concatenate([p16[0] + p16[1], pcnt[0] + pcnt[1]], axis=1)
    wbig = jnp.concatenate(
        [W_e, b_e.reshape(1, D), jnp.zeros((D_AUG - D_E - 1, D), jnp.float32)],
        axis=0)
    return _tc_call(epsilon, nodes, received, aux,
                    wbig, W1, b1.reshape(1, D), W2, b2.reshape(1, D))


def kernel(nodes, edge_attr, senders, receivers, W_e, b_e, epsilon, W1, b1, W2, b2):
    return _impl(nodes, edge_attr, senders, receivers, W_e, b_e, epsilon,
                 W1, b1, W2, b2)
